# baseline (speedup 1.0000x reference)
import jax
import jax.numpy as jnp
from jax.experimental import pallas as pl
from jax.experimental.pallas import tpu as pltpu


def _chunk_sizes():
    sizes = []
    total = 0
    i = 0
    while total < 6400:
        s = 2 + (i % 8)
        sizes.append(s)
        total += s
        i += 1
    return sizes


_SIZES = _chunk_sizes()
_OFFSETS = [0]
for _s in _SIZES:
    _OFFSETS.append(_OFFSETS[-1] + _s)
_N = len(_SIZES)

_NSLAB = 16
_SLAB = 6400 // _NSLAB

_CHUNK_SLAB = [(_OFFSETS[i] + _SIZES[i] - 1) // _SLAB for i in range(_N)]


def _split_body(*refs):
    x_any = refs[0]
    out_refs = refs[1:1 + _N]
    stage = refs[1 + _N:1 + 2 * _N]
    x_vmem = refs[1 + 2 * _N]
    load_sem = refs[2 + 2 * _N]
    out_sem = refs[3 + 2 * _N]

    loads = []
    for g in range(_NSLAB):
        c = pltpu.make_async_copy(
            x_any.at[pl.ds(g * _SLAB, _SLAB), :],
            x_vmem.at[pl.ds(g * _SLAB, _SLAB), :],
            load_sem.at[g])
        c.start()
        loads.append(c)

    out_copies = []
    for g in range(_NSLAB):
        loads[g].wait()
        slab_chunks = [i for i in range(_N) if _CHUNK_SLAB[i] == g]
        slab_chunks.sort(key=lambda i: _OFFSETS[i] % 8 != 0)
        for i in slab_chunks:
            off, s = _OFFSETS[i], _SIZES[i]
            if off % 8 == 0:
                c = pltpu.make_async_copy(
                    x_vmem.at[pl.ds(off, s), :], out_refs[i], out_sem)
            else:
                stage[i][...] = x_vmem[off:off + s, :]
                c = pltpu.make_async_copy(stage[i], out_refs[i], out_sem)
            c.start()
            out_copies.append(c)

    for c in out_copies:
        c.wait()


def kernel(x):
    out_shape = [jax.ShapeDtypeStruct((s, x.shape[1]), x.dtype)
                 for s in _SIZES]
    outs = pl.pallas_call(
        _split_body,
        in_specs=[pl.BlockSpec(memory_space=pl.ANY)],
        out_specs=[pl.BlockSpec(memory_space=pl.ANY)] * _N,
        out_shape=out_shape,
        scratch_shapes=[pltpu.VMEM((s, 512), jnp.float32) for s in _SIZES]
        + [pltpu.VMEM((6400, 512), jnp.float32),
           pltpu.SemaphoreType.DMA((_NSLAB,)),
           pltpu.SemaphoreType.DMA],
    )(x)
    return tuple(outs)


# speedup vs baseline: 1.0064x; 1.0064x over previous
import jax
import jax.numpy as jnp
from jax.experimental import pallas as pl
from jax.experimental.pallas import tpu as pltpu


def _chunk_sizes():
    sizes = []
    total = 0
    i = 0
    while total < 6400:
        s = 2 + (i % 8)
        sizes.append(s)
        total += s
        i += 1
    return sizes


_SIZES = _chunk_sizes()
_OFFSETS = [0]
for _s in _SIZES:
    _OFFSETS.append(_OFFSETS[-1] + _s)
_N = len(_SIZES)

_NSLAB = 4
_SLAB = 6400 // _NSLAB

_CHUNK_SLAB = [(_OFFSETS[i] + _SIZES[i] - 1) // _SLAB for i in range(_N)]


def _split_body(*refs):
    x_any = refs[0]
    out_refs = refs[1:1 + _N]
    stage = refs[1 + _N:1 + 2 * _N]
    x_vmem = refs[1 + 2 * _N]
    load_sem = refs[2 + 2 * _N]
    out_sem = refs[3 + 2 * _N]

    loads = []
    for g in range(_NSLAB):
        c = pltpu.make_async_copy(
            x_any.at[pl.ds(g * _SLAB, _SLAB), :],
            x_vmem.at[pl.ds(g * _SLAB, _SLAB), :],
            load_sem.at[g])
        c.start()
        loads.append(c)

    out_copies = []
    for g in range(_NSLAB):
        loads[g].wait()
        for i in range(_N):
            if _CHUNK_SLAB[i] != g:
                continue
            off, s = _OFFSETS[i], _SIZES[i]
            if off % 8 == 0:
                c = pltpu.make_async_copy(
                    x_vmem.at[pl.ds(off, s), :], out_refs[i], out_sem)
            else:
                stage[i][...] = x_vmem[off:off + s, :]
                c = pltpu.make_async_copy(stage[i], out_refs[i], out_sem)
            c.start()
            out_copies.append(c)

    for c in out_copies:
        c.wait()


def kernel(x):
    out_shape = [jax.ShapeDtypeStruct((s, x.shape[1]), x.dtype)
                 for s in _SIZES]
    outs = pl.pallas_call(
        _split_body,
        in_specs=[pl.BlockSpec(memory_space=pl.ANY)],
        out_specs=[pl.BlockSpec(memory_space=pl.ANY)] * _N,
        out_shape=out_shape,
        scratch_shapes=[pltpu.VMEM((s, 512), jnp.float32) for s in _SIZES]
        + [pltpu.VMEM((6400, 512), jnp.float32),
           pltpu.SemaphoreType.DMA((_NSLAB,)),
           pltpu.SemaphoreType.DMA],
    )(x)
    return tuple(outs)
